# baseline (speedup 1.0000x reference)
import jax
import jax.numpy as jnp
from jax.experimental import pallas as pl
from jax.experimental.pallas import tpu as pltpu
from math import ceil, log2

_DIM = 64
_HEADS = 12
_SIZE = 2048


def _expm_ref(A):
    s = 8
    As = A / (2.0 ** s)
    d = A.shape[-1]
    I = jnp.broadcast_to(jnp.eye(d, dtype=A.dtype), A.shape)
    term = I
    out = I
    for k in range(1, 21):
        term = jnp.matmul(term, As) / float(k)
        out = out + term
    for _ in range(s):
        out = jnp.matmul(out, out)
    return out


def _build_table(primitives):
    herm = primitives - jnp.swapaxes(primitives, -1, -2)
    prim = _expm_ref(herm)
    maps = prim[None]
    for _ in range(ceil(log2(_SIZE))):
        longest = maps[-1]
        expanded = jnp.einsum('nhij,hjk->nhik', maps, longest)
        maps = jnp.concatenate((maps, expanded), axis=0)
    maps = maps[:_SIZE]
    eye = jnp.broadcast_to(
        jnp.eye(_DIM, dtype=primitives.dtype), (_HEADS, _DIM, _DIM))
    return jnp.concatenate((eye[None], maps), axis=0)


def _gather_kernel(ids_ref, tab_ref, out_ref):
    out_ref[...] = tab_ref[...]


def kernel(position_ids, primitives):
    batch, seq = position_ids.shape
    s_total = batch * seq
    ids_flat = position_ids.reshape(s_total).astype(jnp.int32)
    table = _build_table(primitives)

    grid_spec = pltpu.PrefetchScalarGridSpec(
        num_scalar_prefetch=1,
        grid=(s_total,),
        in_specs=[
            pl.BlockSpec((1, _HEADS, _DIM, _DIM), lambda p, ids: (ids[p], 0, 0, 0)),
        ],
        out_specs=pl.BlockSpec(
            (1, _HEADS, _DIM, _DIM), lambda p, ids: (p, 0, 0, 0)),
    )
    out = pl.pallas_call(
        _gather_kernel,
        grid_spec=grid_spec,
        out_shape=jax.ShapeDtypeStruct((s_total, _HEADS, _DIM, _DIM), jnp.float32),
        compiler_params=pltpu.CompilerParams(
            dimension_semantics=("arbitrary",),
        ),
    )(ids_flat, table)

    return out.reshape(batch, seq, _HEADS, _DIM, _DIM)


# speedup vs baseline: 1.2525x; 1.2525x over previous
import functools

import jax
import jax.numpy as jnp
from jax.experimental import pallas as pl
from jax.experimental.pallas import tpu as pltpu

_DIM = 64
_HEADS = 12
_SIZE = 2048
_C = 64
_NTAB = 2112


def _eye(d):
    r = jax.lax.broadcasted_iota(jnp.int32, (d, d), 0)
    c = jax.lax.broadcasted_iota(jnp.int32, (d, d), 1)
    return (r == c).astype(jnp.float32)


def _mm(a, b):
    return jnp.dot(a, b, preferred_element_type=jnp.float32)


def _stage1_kernel(prim_ref, tab_ref, mult_ref):
    p0 = prim_ref[0]
    herm = p0 - p0.T
    a_s = herm * (1.0 / 256.0)
    eye = _eye(_DIM)
    term = eye
    out = eye
    for k in range(1, 21):
        term = _mm(term, a_s) / float(k)
        out = out + term
    for _ in range(8):
        out = _mm(out, out)
    tab_ref[0, 0] = eye
    tab_ref[1, 0] = out
    for n in (1, 2, 4, 8, 16):
        left = tab_ref[1:1 + n, 0].reshape(n * _DIM, _DIM)
        prod = _mm(left, tab_ref[n, 0]).reshape(n, _DIM, _DIM)
        tab_ref[n + 1:2 * n + 1, 0] = prod
    left = tab_ref[1:33, 0].reshape(32 * _DIM, _DIM)
    prod = _mm(left, tab_ref[32, 0]).reshape(32, _DIM, _DIM)
    tab_ref[33:64, 0] = prod[0:31]
    mult_ref[0] = prod[31]


def _step_kernel(tab_ref, mult_ref, out_ref, mult_out_ref, *, n_chunks):
    t = pl.program_id(1)
    m = mult_ref[0]
    mm = _mm(m, m)
    left = tab_ref[:, 0].reshape(_C * _DIM, _DIM)
    prod = _mm(left, m).reshape(_C, _DIM, _DIM)
    row0 = jnp.where(t == 0, m, jnp.where(t == n_chunks, mm, prod[0]))
    out_ref[0, 0] = row0
    out_ref[1:, 0] = prod[1:]
    mult_out_ref[0] = mm


def _step_call(tab, mult, n):
    n_chunks = n // _C
    kern = functools.partial(_step_kernel, n_chunks=n_chunks)
    return pl.pallas_call(
        kern,
        grid=(_HEADS, n_chunks + 1),
        in_specs=[
            pl.BlockSpec(
                (_C, 1, _DIM, _DIM),
                lambda h, t, nc=n_chunks: (jnp.where(t < nc, t, 0), h, 0, 0)),
            pl.BlockSpec((1, _DIM, _DIM), lambda h, t: (h, 0, 0)),
        ],
        out_specs=[
            pl.BlockSpec(
                (_C, 1, _DIM, _DIM),
                lambda h, t, nc=n_chunks: (
                    jnp.where(t < nc, nc + t, 2 * nc), h, 0, 0)),
            pl.BlockSpec((1, _DIM, _DIM), lambda h, t: (h, 0, 0)),
        ],
        out_shape=[
            jax.ShapeDtypeStruct((_NTAB, _HEADS, _DIM, _DIM), jnp.float32),
            jax.ShapeDtypeStruct((_HEADS, _DIM, _DIM), jnp.float32),
        ],
        input_output_aliases={0: 0},
        compiler_params=pltpu.CompilerParams(
            dimension_semantics=("parallel", "arbitrary"),
        ),
    )(tab, mult)


def _gather_kernel(ids_ref, tab_ref, out_ref):
    out_ref[...] = tab_ref[...]


def kernel(position_ids, primitives):
    batch, seq = position_ids.shape
    s_total = batch * seq
    ids_flat = position_ids.reshape(s_total).astype(jnp.int32)

    tab, mult = pl.pallas_call(
        _stage1_kernel,
        grid=(_HEADS,),
        in_specs=[pl.BlockSpec((1, _DIM, _DIM), lambda h: (h, 0, 0))],
        out_specs=[
            pl.BlockSpec((_C, 1, _DIM, _DIM), lambda h: (0, h, 0, 0)),
            pl.BlockSpec((1, _DIM, _DIM), lambda h: (h, 0, 0)),
        ],
        out_shape=[
            jax.ShapeDtypeStruct((_NTAB, _HEADS, _DIM, _DIM), jnp.float32),
            jax.ShapeDtypeStruct((_HEADS, _DIM, _DIM), jnp.float32),
        ],
        compiler_params=pltpu.CompilerParams(
            dimension_semantics=("parallel",),
        ),
    )(primitives)

    for n in (64, 128, 256, 512, 1024):
        tab, mult = _step_call(tab, mult, n)

    grid_spec = pltpu.PrefetchScalarGridSpec(
        num_scalar_prefetch=1,
        grid=(s_total,),
        in_specs=[
            pl.BlockSpec((1, _HEADS, _DIM, _DIM),
                         lambda p, ids: (ids[p], 0, 0, 0)),
        ],
        out_specs=pl.BlockSpec((1, _HEADS, _DIM, _DIM),
                               lambda p, ids: (p, 0, 0, 0)),
    )
    out = pl.pallas_call(
        _gather_kernel,
        grid_spec=grid_spec,
        out_shape=jax.ShapeDtypeStruct(
            (s_total, _HEADS, _DIM, _DIM), jnp.float32),
        compiler_params=pltpu.CompilerParams(
            dimension_semantics=("arbitrary",),
        ),
    )(ids_flat, tab)

    return out.reshape(batch, seq, _HEADS, _DIM, _DIM)
